# Initial kernel scaffold; baseline (speedup 1.0000x reference)
import jax
import jax.numpy as jnp
from jax.experimental import pallas as pl


def kernel(h, x, edge_index, edge_attr, We1, be1, We2, be2, Wx1, bx1, Wx2, Wh1, bh1, Wh2, bh2):
    raise NotImplementedError("write your pallas kernel here")


# speedup vs baseline: 2.9307x; 2.9307x over previous
import jax
import jax.numpy as jnp
from jax import lax
from jax.experimental import pallas as pl
from jax.experimental.pallas import tpu as pltpu
from jax.experimental.pallas import tpu_sc as plsc

N = 10000
E = 320000
ND = 128
HID = 64
XW = 16
TW = HID + XW
LAYERS = 4

NC, NS = 2, 16
NW = NC * NS
EPW = E // NW
CH = 80
NCHUNK = EPW // CH
RPT = N // NS

_mesh = plsc.VectorSubcoreMesh(core_axis_name="c", subcore_axis_name="s")


def _silu(v):
    return v * jax.nn.sigmoid(v)


def _gather_body(ta_hbm, tb_hbm, ii_hbm, jj_hbm, oa_hbm, ob_hbm,
                 iv, jv, bufa, bufb, sema, semb):
    wid = lax.axis_index("s") * NC + lax.axis_index("c")
    base = wid * EPW

    def step(k, carry):
        off = base + k * CH
        pltpu.sync_copy(ii_hbm.at[pl.ds(off, CH)], iv)
        pltpu.sync_copy(jj_hbm.at[pl.ds(off, CH)], jv)
        ca = pltpu.async_copy(ta_hbm.at[iv], bufa, sema)
        cb = pltpu.async_copy(tb_hbm.at[jv], bufb, semb)
        ca.wait()
        cb.wait()
        pltpu.sync_copy(bufa, oa_hbm.at[pl.ds(off, CH)])
        pltpu.sync_copy(bufb, ob_hbm.at[pl.ds(off, CH)])
        return carry

    lax.fori_loop(0, NCHUNK, step, 0)


_sc_params = pltpu.CompilerParams(use_tc_tiling_on_sc=False)

_gather = pl.kernel(
    _gather_body,
    out_type=(jax.ShapeDtypeStruct((E, TW), jnp.float32),
              jax.ShapeDtypeStruct((E, TW), jnp.float32)),
    mesh=_mesh,
    scratch_types=[
        pltpu.VMEM((CH,), jnp.int32),
        pltpu.VMEM((CH,), jnp.int32),
        pltpu.VMEM((CH, TW), jnp.float32),
        pltpu.VMEM((CH, TW), jnp.float32),
        pltpu.SemaphoreType.DMA,
        pltpu.SemaphoreType.DMA,
    ],
    compiler_params=_sc_params,
)


def _scatter_body(m_hbm, ii_hbm, z_hbm, p_hbm, iv, buf, acc, sem):
    c = lax.axis_index("c")
    s = lax.axis_index("s")
    wid = s * NC + c
    pltpu.sync_copy(z_hbm.at[pl.ds(s * RPT, RPT)], acc.at[pl.ds(s * RPT, RPT)])
    plsc.subcore_barrier()

    def step(k, carry):
        off = wid * EPW + k * CH
        pltpu.sync_copy(ii_hbm.at[pl.ds(off, CH)], iv)
        pltpu.sync_copy(m_hbm.at[pl.ds(off, CH)], buf)
        pltpu.sync_copy(buf, acc.at[iv], add=True)
        return carry

    lax.fori_loop(0, NCHUNK, step, 0)
    plsc.subcore_barrier()
    pltpu.sync_copy(acc.at[pl.ds(s * RPT, RPT)], p_hbm.at[c, pl.ds(s * RPT, RPT)])


_scatter = pl.kernel(
    _scatter_body,
    out_type=jax.ShapeDtypeStruct((NC, N, TW), jnp.float32),
    mesh=_mesh,
    scratch_types=[
        pltpu.VMEM((CH,), jnp.int32),
        pltpu.VMEM((CH, TW), jnp.float32),
        pltpu.VMEM_SHARED((N, TW), jnp.float32),
        pltpu.SemaphoreType.DMA,
    ],
    compiler_params=_sc_params,
)


BN = 2000


def _node_pre_body(h_ref, x_ref, wa_ref, wb_ref, be1_ref, ta_ref, tb_ref):
    h = h_ref[...]
    x = x_ref[...]
    a = jnp.dot(h, wa_ref[...], preferred_element_type=jnp.float32) + be1_ref[...]
    b = jnp.dot(h, wb_ref[...], preferred_element_type=jnp.float32)
    ta_ref[...] = jnp.concatenate([a, x], axis=1)
    tb_ref[...] = jnp.concatenate([b, -x], axis=1)


_node_pre = pl.pallas_call(
    _node_pre_body,
    grid=(N // BN,),
    in_specs=[
        pl.BlockSpec((BN, ND), lambda i: (i, 0)),
        pl.BlockSpec((BN, XW), lambda i: (i, 0)),
        pl.BlockSpec((ND, HID), lambda i: (0, 0)),
        pl.BlockSpec((ND, HID), lambda i: (0, 0)),
        pl.BlockSpec((1, HID), lambda i: (0, 0)),
    ],
    out_specs=(pl.BlockSpec((BN, TW), lambda i: (i, 0)),
               pl.BlockSpec((BN, TW), lambda i: (i, 0))),
    out_shape=(jax.ShapeDtypeStruct((N, TW), jnp.float32),
               jax.ShapeDtypeStruct((N, TW), jnp.float32)),
)


BE = 2560


def _edge_mlp_body(a_ref, b_ref, ea_ref, wea_ref, w256_ref, we2_ref, be2_ref,
                   wx1_ref, bx1_ref, wx2_ref, m_ref):
    a = a_ref[...]
    b = b_ref[...]
    s = a[:, :HID] + b[:, :HID]
    d = a[:, HID:] + b[:, HID:]
    dsq = jnp.sum(d * d, axis=1, keepdims=True)
    dsqb = dsq.astype(jnp.bfloat16).astype(jnp.float32)
    w256b = w256_ref[...].astype(jnp.bfloat16).astype(jnp.float32)
    pre = (s + dsqb * w256b
           + jnp.dot(ea_ref[...], wea_ref[...], preferred_element_type=jnp.float32))
    m1 = _silu(pre)
    m2 = _silu(jnp.dot(m1, we2_ref[...], preferred_element_type=jnp.float32)
               + be2_ref[...])
    t = _silu(jnp.dot(m2, wx1_ref[...], preferred_element_type=jnp.float32)
              + bx1_ref[...])
    tb16 = t.astype(jnp.bfloat16).astype(jnp.float32)
    wxb = wx2_ref[...].astype(jnp.bfloat16).astype(jnp.float32)
    cw = jnp.sum(tb16 * wxb, axis=1, keepdims=True)
    m_ref[...] = jnp.concatenate([m2, d * cw], axis=1)


_edge_mlp = pl.pallas_call(
    _edge_mlp_body,
    grid=(E // BE,),
    in_specs=[
        pl.BlockSpec((BE, TW), lambda i: (i, 0)),
        pl.BlockSpec((BE, TW), lambda i: (i, 0)),
        pl.BlockSpec((BE, XW), lambda i: (i, 0)),
        pl.BlockSpec((XW, HID), lambda i: (0, 0)),
        pl.BlockSpec((1, HID), lambda i: (0, 0)),
        pl.BlockSpec((HID, HID), lambda i: (0, 0)),
        pl.BlockSpec((1, HID), lambda i: (0, 0)),
        pl.BlockSpec((HID, HID), lambda i: (0, 0)),
        pl.BlockSpec((1, HID), lambda i: (0, 0)),
        pl.BlockSpec((1, HID), lambda i: (0, 0)),
    ],
    out_specs=pl.BlockSpec((BE, TW), lambda i: (i, 0)),
    out_shape=jax.ShapeDtypeStruct((E, TW), jnp.float32),
)


def _node_upd_body(h_ref, x_ref, p0_ref, p1_ref, w1h_ref, w1m_ref, bh1_ref,
                   wh2_ref, bh2_ref, ho_ref, xo_ref):
    h = h_ref[...]
    p0 = p0_ref[...]
    p1 = p1_ref[...]
    mi = p0[:, :HID] + p1[:, :HID]
    xu = p0[:, HID:] + p1[:, HID:]
    g = _silu(jnp.dot(h, w1h_ref[...], preferred_element_type=jnp.float32)
              + jnp.dot(mi, w1m_ref[...], preferred_element_type=jnp.float32)
              + bh1_ref[...])
    ho_ref[...] = h + jnp.dot(g, wh2_ref[...], preferred_element_type=jnp.float32) + bh2_ref[...]
    xo_ref[...] = x_ref[...] + xu


_node_upd = pl.pallas_call(
    _node_upd_body,
    grid=(N // BN,),
    in_specs=[
        pl.BlockSpec((BN, ND), lambda i: (i, 0)),
        pl.BlockSpec((BN, XW), lambda i: (i, 0)),
        pl.BlockSpec((BN, TW), lambda i: (i, 0)),
        pl.BlockSpec((BN, TW), lambda i: (i, 0)),
        pl.BlockSpec((ND, HID), lambda i: (0, 0)),
        pl.BlockSpec((HID, HID), lambda i: (0, 0)),
        pl.BlockSpec((1, HID), lambda i: (0, 0)),
        pl.BlockSpec((HID, ND), lambda i: (0, 0)),
        pl.BlockSpec((1, ND), lambda i: (0, 0)),
    ],
    out_specs=(pl.BlockSpec((BN, ND), lambda i: (i, 0)),
               pl.BlockSpec((BN, XW), lambda i: (i, 0))),
    out_shape=(jax.ShapeDtypeStruct((N, ND), jnp.float32),
               jax.ShapeDtypeStruct((N, XW), jnp.float32)),
)


def kernel(h, x, edge_index, edge_attr, We1, be1, We2, be2, Wx1, bx1, Wx2,
           Wh1, bh1, Wh2, bh2):
    ii = edge_index[1].astype(jnp.int32)
    jj = edge_index[0].astype(jnp.int32)
    xp = jnp.pad(x, ((0, 0), (0, XW - 3)))
    zacc = jnp.zeros((N, TW), jnp.float32)
    for l in range(LAYERS):
        wa = We1[l, :ND]
        wb = We1[l, ND:2 * ND]
        w256 = We1[l, 2 * ND:2 * ND + 1]
        wea = We1[l, 2 * ND + 1:]
        ta, tb = _node_pre(h, xp, wa, wb, be1[l][None])
        ga, gb = _gather(ta, tb, ii, jj)
        m = _edge_mlp(ga, gb, edge_attr, wea, w256, We2[l], be2[l][None],
                      Wx1[l], bx1[l][None], Wx2[l].T)
        p = _scatter(m, ii, zacc)
        h, xp = _node_upd(h, xp, p[0], p[1], Wh1[l, :ND], Wh1[l, ND:],
                          bh1[l][None], Wh2[l], bh2[l][None])
    return h, xp[:, :3]


# speedup vs baseline: 3.9736x; 1.3558x over previous
import jax
import jax.numpy as jnp
from jax import lax
from jax.experimental import pallas as pl
from jax.experimental.pallas import tpu as pltpu
from jax.experimental.pallas import tpu_sc as plsc

N = 10000
E = 320000
ND = 128
HID = 64
XW = 16
TW = HID + XW
LAYERS = 4

NC, NS = 2, 16
NW = NC * NS
EPW = E // NW
CH = 80
NCHUNK = EPW // CH
RPT = N // NS

_mesh = plsc.VectorSubcoreMesh(core_axis_name="c", subcore_axis_name="s")


def _silu(v):
    return v * jax.nn.sigmoid(v)


SCH = 200
QCH = 40
NQ = SCH // QCH
NSUP = EPW // SCH
NHALF = NSUP // 2


def _gather_start(ta_hbm, tb_hbm, ii_hbm, jj_hbm, off, iv, jv, bufa, bufb,
                  sa, sb):
    pltpu.sync_copy(ii_hbm.at[pl.ds(off, SCH)], iv)
    pltpu.sync_copy(jj_hbm.at[pl.ds(off, SCH)], jv)
    for q in range(NQ):
        r = pl.ds(q * QCH, QCH)
        pltpu.async_copy(ta_hbm.at[iv.at[r]], bufa.at[r], sa)
        pltpu.async_copy(tb_hbm.at[jv.at[r]], bufb.at[r], sb)


def _gather_finish(og_hbm, off, iv, jv, bufa, bufb, sa, sb):
    for q in range(NQ):
        r = pl.ds(q * QCH, QCH)
        pltpu.make_async_copy(og_hbm.at[pl.ds(0, QCH)], bufa.at[r], sa).wait()
        pltpu.make_async_copy(og_hbm.at[pl.ds(0, QCH)], bufb.at[r], sb).wait()

    def add_row(rr, carry):
        for c in range(TW // 16):
            cs = pl.ds(c * 16, 16)
            plsc.addupdate(bufa.at[rr, cs], bufb[rr, cs])
        return carry

    lax.fori_loop(0, SCH, add_row, 0)
    pltpu.sync_copy(bufa, og_hbm.at[pl.ds(off, SCH)])


def _gather_body(ta_hbm, tb_hbm, ii_hbm, jj_hbm, og_hbm,
                 iv0, jv0, iv1, jv1, bufa0, bufb0, bufa1, bufb1,
                 sa0, sb0, sa1, sb1):
    wid = lax.axis_index("s") * NC + lax.axis_index("c")
    base = wid * EPW
    _gather_start(ta_hbm, tb_hbm, ii_hbm, jj_hbm, base, iv0, jv0,
                  bufa0, bufb0, sa0, sb0)

    def outer(g, carry):
        off_b = base + (2 * g + 1) * SCH
        _gather_start(ta_hbm, tb_hbm, ii_hbm, jj_hbm, off_b, iv1, jv1,
                      bufa1, bufb1, sa1, sb1)
        _gather_finish(og_hbm, base + 2 * g * SCH, iv0, jv0, bufa0, bufb0,
                       sa0, sb0)

        @pl.when(g < NHALF - 1)
        def _():
            _gather_start(ta_hbm, tb_hbm, ii_hbm, jj_hbm,
                          base + (2 * g + 2) * SCH, iv0, jv0,
                          bufa0, bufb0, sa0, sb0)

        _gather_finish(og_hbm, off_b, iv1, jv1, bufa1, bufb1, sa1, sb1)
        return carry

    lax.fori_loop(0, NHALF, outer, 0)


_sc_params = pltpu.CompilerParams(use_tc_tiling_on_sc=False)

_gather = pl.kernel(
    _gather_body,
    out_type=jax.ShapeDtypeStruct((E, TW), jnp.float32),
    mesh=_mesh,
    scratch_types=[
        pltpu.VMEM((SCH,), jnp.int32),
        pltpu.VMEM((SCH,), jnp.int32),
        pltpu.VMEM((SCH,), jnp.int32),
        pltpu.VMEM((SCH,), jnp.int32),
        pltpu.VMEM((SCH, TW), jnp.float32),
        pltpu.VMEM((SCH, TW), jnp.float32),
        pltpu.VMEM((SCH, TW), jnp.float32),
        pltpu.VMEM((SCH, TW), jnp.float32),
        pltpu.SemaphoreType.DMA,
        pltpu.SemaphoreType.DMA,
        pltpu.SemaphoreType.DMA,
        pltpu.SemaphoreType.DMA,
    ],
    compiler_params=_sc_params,
)


def _scatter_body(m_hbm, ii_hbm, z_hbm, p_hbm, iv, buf, acc, sem):
    c = lax.axis_index("c")
    s = lax.axis_index("s")
    wid = s * NC + c
    pltpu.sync_copy(z_hbm.at[pl.ds(s * RPT, RPT)], acc.at[pl.ds(s * RPT, RPT)])
    plsc.subcore_barrier()

    def step(k, carry):
        off = wid * EPW + k * CH
        pltpu.sync_copy(ii_hbm.at[pl.ds(off, CH)], iv)
        pltpu.sync_copy(m_hbm.at[pl.ds(off, CH)], buf)
        pltpu.sync_copy(buf, acc.at[iv], add=True)
        return carry

    lax.fori_loop(0, NCHUNK, step, 0)
    plsc.subcore_barrier()
    pltpu.sync_copy(acc.at[pl.ds(s * RPT, RPT)], p_hbm.at[c, pl.ds(s * RPT, RPT)])


_scatter = pl.kernel(
    _scatter_body,
    out_type=jax.ShapeDtypeStruct((NC, N, TW), jnp.float32),
    mesh=_mesh,
    scratch_types=[
        pltpu.VMEM((CH,), jnp.int32),
        pltpu.VMEM((CH, TW), jnp.float32),
        pltpu.VMEM_SHARED((N, TW), jnp.float32),
        pltpu.SemaphoreType.DMA,
    ],
    compiler_params=_sc_params,
)


BN = 2000


def _node_pre_body(h_ref, x_ref, wa_ref, wb_ref, be1_ref, ta_ref, tb_ref):
    h = h_ref[...]
    x = x_ref[...]
    a = jnp.dot(h, wa_ref[...], preferred_element_type=jnp.float32) + be1_ref[...]
    b = jnp.dot(h, wb_ref[...], preferred_element_type=jnp.float32)
    ta_ref[...] = jnp.concatenate([a, x], axis=1)
    tb_ref[...] = jnp.concatenate([b, -x], axis=1)


_node_pre = pl.pallas_call(
    _node_pre_body,
    grid=(N // BN,),
    in_specs=[
        pl.BlockSpec((BN, ND), lambda i: (i, 0)),
        pl.BlockSpec((BN, XW), lambda i: (i, 0)),
        pl.BlockSpec((ND, HID), lambda i: (0, 0)),
        pl.BlockSpec((ND, HID), lambda i: (0, 0)),
        pl.BlockSpec((1, HID), lambda i: (0, 0)),
    ],
    out_specs=(pl.BlockSpec((BN, TW), lambda i: (i, 0)),
               pl.BlockSpec((BN, TW), lambda i: (i, 0))),
    out_shape=(jax.ShapeDtypeStruct((N, TW), jnp.float32),
               jax.ShapeDtypeStruct((N, TW), jnp.float32)),
)


BE = 2560


def _edge_mlp_body(g_ref, ea_ref, wea_ref, w256_ref, we2_ref, be2_ref,
                   wx1_ref, bx1_ref, wx2_ref, m_ref):
    g = g_ref[...]
    s = g[:, :HID]
    d = g[:, HID:]
    dsq = jnp.sum(d * d, axis=1, keepdims=True)
    dsqb = dsq.astype(jnp.bfloat16).astype(jnp.float32)
    w256b = w256_ref[...].astype(jnp.bfloat16).astype(jnp.float32)
    pre = (s + dsqb * w256b
           + jnp.dot(ea_ref[...], wea_ref[...], preferred_element_type=jnp.float32))
    m1 = _silu(pre)
    m2 = _silu(jnp.dot(m1, we2_ref[...], preferred_element_type=jnp.float32)
               + be2_ref[...])
    t = _silu(jnp.dot(m2, wx1_ref[...], preferred_element_type=jnp.float32)
              + bx1_ref[...])
    tb16 = t.astype(jnp.bfloat16).astype(jnp.float32)
    wxb = wx2_ref[...].astype(jnp.bfloat16).astype(jnp.float32)
    cw = jnp.sum(tb16 * wxb, axis=1, keepdims=True)
    m_ref[...] = jnp.concatenate([m2, d * cw], axis=1)


_edge_mlp = pl.pallas_call(
    _edge_mlp_body,
    grid=(E // BE,),
    in_specs=[
        pl.BlockSpec((BE, TW), lambda i: (i, 0)),
        pl.BlockSpec((BE, XW), lambda i: (i, 0)),
        pl.BlockSpec((XW, HID), lambda i: (0, 0)),
        pl.BlockSpec((1, HID), lambda i: (0, 0)),
        pl.BlockSpec((HID, HID), lambda i: (0, 0)),
        pl.BlockSpec((1, HID), lambda i: (0, 0)),
        pl.BlockSpec((HID, HID), lambda i: (0, 0)),
        pl.BlockSpec((1, HID), lambda i: (0, 0)),
        pl.BlockSpec((1, HID), lambda i: (0, 0)),
    ],
    out_specs=pl.BlockSpec((BE, TW), lambda i: (i, 0)),
    out_shape=jax.ShapeDtypeStruct((E, TW), jnp.float32),
)


def _node_upd_body(h_ref, x_ref, p0_ref, p1_ref, w1h_ref, w1m_ref, bh1_ref,
                   wh2_ref, bh2_ref, ho_ref, xo_ref):
    h = h_ref[...]
    p0 = p0_ref[...]
    p1 = p1_ref[...]
    mi = p0[:, :HID] + p1[:, :HID]
    xu = p0[:, HID:] + p1[:, HID:]
    g = _silu(jnp.dot(h, w1h_ref[...], preferred_element_type=jnp.float32)
              + jnp.dot(mi, w1m_ref[...], preferred_element_type=jnp.float32)
              + bh1_ref[...])
    ho_ref[...] = h + jnp.dot(g, wh2_ref[...], preferred_element_type=jnp.float32) + bh2_ref[...]
    xo_ref[...] = x_ref[...] + xu


_node_upd = pl.pallas_call(
    _node_upd_body,
    grid=(N // BN,),
    in_specs=[
        pl.BlockSpec((BN, ND), lambda i: (i, 0)),
        pl.BlockSpec((BN, XW), lambda i: (i, 0)),
        pl.BlockSpec((BN, TW), lambda i: (i, 0)),
        pl.BlockSpec((BN, TW), lambda i: (i, 0)),
        pl.BlockSpec((ND, HID), lambda i: (0, 0)),
        pl.BlockSpec((HID, HID), lambda i: (0, 0)),
        pl.BlockSpec((1, HID), lambda i: (0, 0)),
        pl.BlockSpec((HID, ND), lambda i: (0, 0)),
        pl.BlockSpec((1, ND), lambda i: (0, 0)),
    ],
    out_specs=(pl.BlockSpec((BN, ND), lambda i: (i, 0)),
               pl.BlockSpec((BN, XW), lambda i: (i, 0))),
    out_shape=(jax.ShapeDtypeStruct((N, ND), jnp.float32),
               jax.ShapeDtypeStruct((N, XW), jnp.float32)),
)


def kernel(h, x, edge_index, edge_attr, We1, be1, We2, be2, Wx1, bx1, Wx2,
           Wh1, bh1, Wh2, bh2):
    ii = edge_index[1].astype(jnp.int32)
    jj = edge_index[0].astype(jnp.int32)
    xp = jnp.pad(x, ((0, 0), (0, XW - 3)))
    zacc = jnp.zeros((N, TW), jnp.float32)
    for l in range(LAYERS):
        wa = We1[l, :ND]
        wb = We1[l, ND:2 * ND]
        w256 = We1[l, 2 * ND:2 * ND + 1]
        wea = We1[l, 2 * ND + 1:]
        ta, tb = _node_pre(h, xp, wa, wb, be1[l][None])
        gg = _gather(ta, tb, ii, jj)
        m = _edge_mlp(gg, edge_attr, wea, w256, We2[l], be2[l][None],
                      Wx1[l], bx1[l][None], Wx2[l].T)
        p = _scatter(m, ii, zacc)
        h, xp = _node_upd(h, xp, p[0], p[1], Wh1[l, :ND], Wh1[l, ND:],
                          bh1[l][None], Wh2[l], bh2[l][None])
    return h, xp[:, :3]


# speedup vs baseline: 4.5154x; 1.1364x over previous
import jax
import jax.numpy as jnp
from jax import lax
from jax.experimental import pallas as pl
from jax.experimental.pallas import tpu as pltpu
from jax.experimental.pallas import tpu_sc as plsc

N = 10000
E = 320000
ND = 128
HID = 64
XW = 16
TW = HID + XW
LAYERS = 4

NC, NS = 2, 16
NW = NC * NS
EPW = E // NW
CH = 80
NCHUNK = EPW // CH
RPT = N // NS

_mesh = plsc.VectorSubcoreMesh(core_axis_name="c", subcore_axis_name="s")


def _silu(v):
    return v * jax.nn.sigmoid(v)


SCH = 200
QCH = 40
NQ = SCH // QCH
NSUP = EPW // SCH
NHALF = NSUP // 2


def _gather_start(ta_hbm, tb_hbm, ii_hbm, jj_hbm, off, iv, jv, bufa, bufb,
                  sa, sb):
    pltpu.sync_copy(ii_hbm.at[pl.ds(off, SCH)], iv)
    pltpu.sync_copy(jj_hbm.at[pl.ds(off, SCH)], jv)
    for q in range(NQ):
        r = pl.ds(q * QCH, QCH)
        pltpu.async_copy(ta_hbm.at[iv.at[r]], bufa.at[r], sa)
        pltpu.async_copy(tb_hbm.at[jv.at[r]], bufb.at[r], sb)


def _gather_finish(og_hbm, off, iv, jv, bufa, bufb, sa, sb):
    for q in range(NQ):
        r = pl.ds(q * QCH, QCH)
        pltpu.make_async_copy(og_hbm.at[pl.ds(0, QCH)], bufa.at[r], sa).wait()
        pltpu.make_async_copy(og_hbm.at[pl.ds(0, QCH)], bufb.at[r], sb).wait()

    def add_row(rr, carry):
        for c in range(TW // 16):
            cs = pl.ds(c * 16, 16)
            plsc.addupdate(bufa.at[rr, cs], bufb[rr, cs])
        return carry

    lax.fori_loop(0, SCH, add_row, 0)
    pltpu.sync_copy(bufa, og_hbm.at[pl.ds(off, SCH)])


def _gather_body(ta_hbm, tb_hbm, ii_hbm, jj_hbm, og_hbm,
                 iv0, jv0, iv1, jv1, bufa0, bufb0, bufa1, bufb1,
                 sa0, sb0, sa1, sb1):
    wid = lax.axis_index("s") * NC + lax.axis_index("c")
    base = wid * EPW
    _gather_start(ta_hbm, tb_hbm, ii_hbm, jj_hbm, base, iv0, jv0,
                  bufa0, bufb0, sa0, sb0)

    def outer(g, carry):
        off_b = base + (2 * g + 1) * SCH
        _gather_start(ta_hbm, tb_hbm, ii_hbm, jj_hbm, off_b, iv1, jv1,
                      bufa1, bufb1, sa1, sb1)
        _gather_finish(og_hbm, base + 2 * g * SCH, iv0, jv0, bufa0, bufb0,
                       sa0, sb0)

        @pl.when(g < NHALF - 1)
        def _():
            _gather_start(ta_hbm, tb_hbm, ii_hbm, jj_hbm,
                          base + (2 * g + 2) * SCH, iv0, jv0,
                          bufa0, bufb0, sa0, sb0)

        _gather_finish(og_hbm, off_b, iv1, jv1, bufa1, bufb1, sa1, sb1)
        return carry

    lax.fori_loop(0, NHALF, outer, 0)


_sc_params = pltpu.CompilerParams(use_tc_tiling_on_sc=False)

_gather = pl.kernel(
    _gather_body,
    out_type=jax.ShapeDtypeStruct((E, TW), jnp.float32),
    mesh=_mesh,
    scratch_types=[
        pltpu.VMEM((SCH,), jnp.int32),
        pltpu.VMEM((SCH,), jnp.int32),
        pltpu.VMEM((SCH,), jnp.int32),
        pltpu.VMEM((SCH,), jnp.int32),
        pltpu.VMEM((SCH, TW), jnp.float32),
        pltpu.VMEM((SCH, TW), jnp.float32),
        pltpu.VMEM((SCH, TW), jnp.float32),
        pltpu.VMEM((SCH, TW), jnp.float32),
        pltpu.SemaphoreType.DMA,
        pltpu.SemaphoreType.DMA,
        pltpu.SemaphoreType.DMA,
        pltpu.SemaphoreType.DMA,
    ],
    compiler_params=_sc_params,
)


SD = 5
SGRP = NCHUNK // SD


def _scat_start(m_hbm, ii_hbm, off, iv, buf, sem):
    pltpu.async_copy(ii_hbm.at[pl.ds(off, CH)], iv, sem)
    pltpu.async_copy(m_hbm.at[pl.ds(off, CH)], buf, sem)


def _scat_flush(m_hbm, ii_hbm, acc, iv, buf, sem):
    pltpu.make_async_copy(ii_hbm.at[pl.ds(0, CH)], iv, sem).wait()
    pltpu.make_async_copy(m_hbm.at[pl.ds(0, CH)], buf, sem).wait()
    pltpu.sync_copy(buf, acc.at[iv], add=True)


def _scatter_body(m_hbm, ii_hbm, z_hbm, p_hbm,
                  iv0, iv1, iv2, iv3, iv4, b0, b1, b2, b3, b4, acc,
                  s0, s1, s2, s3, s4):
    c = lax.axis_index("c")
    s = lax.axis_index("s")
    wid = s * NC + c
    base = wid * EPW
    ivs = (iv0, iv1, iv2, iv3, iv4)
    bufs = (b0, b1, b2, b3, b4)
    sems = (s0, s1, s2, s3, s4)
    pltpu.sync_copy(z_hbm.at[pl.ds(s * RPT, RPT)], acc.at[pl.ds(s * RPT, RPT)])
    plsc.subcore_barrier()
    _scat_start(m_hbm, ii_hbm, base, ivs[0], bufs[0], sems[0])

    def outer(g, carry):
        for q in range(SD):
            nxt = SD * g + q + 1
            qn = (q + 1) % SD

            @pl.when(nxt < NCHUNK)
            def _():
                _scat_start(m_hbm, ii_hbm, base + nxt * CH,
                            ivs[qn], bufs[qn], sems[qn])

            _scat_flush(m_hbm, ii_hbm, acc, ivs[q], bufs[q], sems[q])
        return carry

    lax.fori_loop(0, SGRP, outer, 0)
    plsc.subcore_barrier()
    pltpu.sync_copy(acc.at[pl.ds(s * RPT, RPT)], p_hbm.at[c, pl.ds(s * RPT, RPT)])


_scatter = pl.kernel(
    _scatter_body,
    out_type=jax.ShapeDtypeStruct((NC, N, TW), jnp.float32),
    mesh=_mesh,
    scratch_types=(
        [pltpu.VMEM((CH,), jnp.int32)] * SD
        + [pltpu.VMEM((CH, TW), jnp.float32)] * SD
        + [pltpu.VMEM_SHARED((N, TW), jnp.float32)]
        + [pltpu.SemaphoreType.DMA] * SD
    ),
    compiler_params=_sc_params,
)


BN = 2000


def _node_pre_body(h_ref, x_ref, wa_ref, wb_ref, be1_ref, ta_ref, tb_ref):
    h = h_ref[...]
    x = x_ref[...]
    a = jnp.dot(h, wa_ref[...], preferred_element_type=jnp.float32) + be1_ref[...]
    b = jnp.dot(h, wb_ref[...], preferred_element_type=jnp.float32)
    ta_ref[...] = jnp.concatenate([a, x], axis=1)
    tb_ref[...] = jnp.concatenate([b, -x], axis=1)


_node_pre = pl.pallas_call(
    _node_pre_body,
    grid=(N // BN,),
    in_specs=[
        pl.BlockSpec((BN, ND), lambda i: (i, 0)),
        pl.BlockSpec((BN, XW), lambda i: (i, 0)),
        pl.BlockSpec((ND, HID), lambda i: (0, 0)),
        pl.BlockSpec((ND, HID), lambda i: (0, 0)),
        pl.BlockSpec((1, HID), lambda i: (0, 0)),
    ],
    out_specs=(pl.BlockSpec((BN, TW), lambda i: (i, 0)),
               pl.BlockSpec((BN, TW), lambda i: (i, 0))),
    out_shape=(jax.ShapeDtypeStruct((N, TW), jnp.float32),
               jax.ShapeDtypeStruct((N, TW), jnp.float32)),
)


BE = 2560


def _edge_mlp_body(g_ref, ea_ref, wea_ref, w256_ref, we2_ref, be2_ref,
                   wx1_ref, bx1_ref, wx2_ref, m_ref):
    g = g_ref[...]
    s = g[:, :HID]
    d = g[:, HID:]
    dsq = jnp.sum(d * d, axis=1, keepdims=True)
    dsqb = dsq.astype(jnp.bfloat16).astype(jnp.float32)
    w256b = w256_ref[...].astype(jnp.bfloat16).astype(jnp.float32)
    pre = (s + dsqb * w256b
           + jnp.dot(ea_ref[...], wea_ref[...], preferred_element_type=jnp.float32))
    m1 = _silu(pre)
    m2 = _silu(jnp.dot(m1, we2_ref[...], preferred_element_type=jnp.float32)
               + be2_ref[...])
    t = _silu(jnp.dot(m2, wx1_ref[...], preferred_element_type=jnp.float32)
              + bx1_ref[...])
    tb16 = t.astype(jnp.bfloat16).astype(jnp.float32)
    wxb = wx2_ref[...].astype(jnp.bfloat16).astype(jnp.float32)
    cw = jnp.sum(tb16 * wxb, axis=1, keepdims=True)
    m_ref[...] = jnp.concatenate([m2, d * cw], axis=1)


_edge_mlp = pl.pallas_call(
    _edge_mlp_body,
    grid=(E // BE,),
    in_specs=[
        pl.BlockSpec((BE, TW), lambda i: (i, 0)),
        pl.BlockSpec((BE, XW), lambda i: (i, 0)),
        pl.BlockSpec((XW, HID), lambda i: (0, 0)),
        pl.BlockSpec((1, HID), lambda i: (0, 0)),
        pl.BlockSpec((HID, HID), lambda i: (0, 0)),
        pl.BlockSpec((1, HID), lambda i: (0, 0)),
        pl.BlockSpec((HID, HID), lambda i: (0, 0)),
        pl.BlockSpec((1, HID), lambda i: (0, 0)),
        pl.BlockSpec((1, HID), lambda i: (0, 0)),
    ],
    out_specs=pl.BlockSpec((BE, TW), lambda i: (i, 0)),
    out_shape=jax.ShapeDtypeStruct((E, TW), jnp.float32),
)


def _node_upd_body(h_ref, x_ref, p0_ref, p1_ref, w1h_ref, w1m_ref, bh1_ref,
                   wh2_ref, bh2_ref, ho_ref, xo_ref):
    h = h_ref[...]
    p0 = p0_ref[...]
    p1 = p1_ref[...]
    mi = p0[:, :HID] + p1[:, :HID]
    xu = p0[:, HID:] + p1[:, HID:]
    g = _silu(jnp.dot(h, w1h_ref[...], preferred_element_type=jnp.float32)
              + jnp.dot(mi, w1m_ref[...], preferred_element_type=jnp.float32)
              + bh1_ref[...])
    ho_ref[...] = h + jnp.dot(g, wh2_ref[...], preferred_element_type=jnp.float32) + bh2_ref[...]
    xo_ref[...] = x_ref[...] + xu


_node_upd = pl.pallas_call(
    _node_upd_body,
    grid=(N // BN,),
    in_specs=[
        pl.BlockSpec((BN, ND), lambda i: (i, 0)),
        pl.BlockSpec((BN, XW), lambda i: (i, 0)),
        pl.BlockSpec((BN, TW), lambda i: (i, 0)),
        pl.BlockSpec((BN, TW), lambda i: (i, 0)),
        pl.BlockSpec((ND, HID), lambda i: (0, 0)),
        pl.BlockSpec((HID, HID), lambda i: (0, 0)),
        pl.BlockSpec((1, HID), lambda i: (0, 0)),
        pl.BlockSpec((HID, ND), lambda i: (0, 0)),
        pl.BlockSpec((1, ND), lambda i: (0, 0)),
    ],
    out_specs=(pl.BlockSpec((BN, ND), lambda i: (i, 0)),
               pl.BlockSpec((BN, XW), lambda i: (i, 0))),
    out_shape=(jax.ShapeDtypeStruct((N, ND), jnp.float32),
               jax.ShapeDtypeStruct((N, XW), jnp.float32)),
)


def kernel(h, x, edge_index, edge_attr, We1, be1, We2, be2, Wx1, bx1, Wx2,
           Wh1, bh1, Wh2, bh2):
    ii = edge_index[1].astype(jnp.int32)
    jj = edge_index[0].astype(jnp.int32)
    xp = jnp.pad(x, ((0, 0), (0, XW - 3)))
    zacc = jnp.zeros((N, TW), jnp.float32)
    for l in range(LAYERS):
        wa = We1[l, :ND]
        wb = We1[l, ND:2 * ND]
        w256 = We1[l, 2 * ND:2 * ND + 1]
        wea = We1[l, 2 * ND + 1:]
        ta, tb = _node_pre(h, xp, wa, wb, be1[l][None])
        gg = _gather(ta, tb, ii, jj)
        m = _edge_mlp(gg, edge_attr, wea, w256, We2[l], be2[l][None],
                      Wx1[l], bx1[l][None], Wx2[l].T)
        p = _scatter(m, ii, zacc)
        h, xp = _node_upd(h, xp, p[0], p[1], Wh1[l, :ND], Wh1[l, ND:],
                          bh1[l][None], Wh2[l], bh2[l][None])
    return h, xp[:, :3]


# speedup vs baseline: 4.5472x; 1.0070x over previous
import jax
import jax.numpy as jnp
from jax import lax
from jax.experimental import pallas as pl
from jax.experimental.pallas import tpu as pltpu
from jax.experimental.pallas import tpu_sc as plsc

N = 10000
E = 320000
ND = 128
HID = 64
XW = 16
TW = HID + XW
LAYERS = 4

NC, NS = 2, 16
NW = NC * NS
EPW = E // NW
CH = 80
NCHUNK = EPW // CH
RPT = N // NS

_mesh = plsc.VectorSubcoreMesh(core_axis_name="c", subcore_axis_name="s")


def _silu(v):
    return v * jax.nn.sigmoid(v)


SCH = 200
QCH = 40
NQ = SCH // QCH
NSUP = EPW // SCH
NHALF = NSUP // 2


def _gather_start(ta_hbm, tb_hbm, ii_hbm, jj_hbm, off, iv, jv, bufa, bufb,
                  sa, sb):
    pltpu.sync_copy(ii_hbm.at[pl.ds(off, SCH)], iv)
    pltpu.sync_copy(jj_hbm.at[pl.ds(off, SCH)], jv)
    for q in range(NQ):
        r = pl.ds(q * QCH, QCH)
        pltpu.async_copy(ta_hbm.at[iv.at[r]], bufa.at[r], sa)
        pltpu.async_copy(tb_hbm.at[jv.at[r]], bufb.at[r], sb)


def _gather_finish(og_hbm, off, iv, jv, bufa, bufb, sa, sb):
    for q in range(NQ):
        r = pl.ds(q * QCH, QCH)
        pltpu.make_async_copy(og_hbm.at[pl.ds(0, QCH)], bufa.at[r], sa).wait()
        pltpu.make_async_copy(og_hbm.at[pl.ds(0, QCH)], bufb.at[r], sb).wait()

    def add_row(rr, carry):
        for c in range(TW // 16):
            cs = pl.ds(c * 16, 16)
            plsc.addupdate(bufa.at[rr, cs], bufb[rr, cs])
        return carry

    lax.fori_loop(0, SCH, add_row, 0)
    pltpu.sync_copy(bufa, og_hbm.at[pl.ds(off, SCH)])


def _gather_body(ta_hbm, tb_hbm, ii_hbm, jj_hbm, og_hbm,
                 iv0, jv0, iv1, jv1, bufa0, bufb0, bufa1, bufb1,
                 sa0, sb0, sa1, sb1):
    wid = lax.axis_index("s") * NC + lax.axis_index("c")
    base = wid * EPW
    _gather_start(ta_hbm, tb_hbm, ii_hbm, jj_hbm, base, iv0, jv0,
                  bufa0, bufb0, sa0, sb0)

    def outer(g, carry):
        off_b = base + (2 * g + 1) * SCH
        _gather_start(ta_hbm, tb_hbm, ii_hbm, jj_hbm, off_b, iv1, jv1,
                      bufa1, bufb1, sa1, sb1)
        _gather_finish(og_hbm, base + 2 * g * SCH, iv0, jv0, bufa0, bufb0,
                       sa0, sb0)

        @pl.when(g < NHALF - 1)
        def _():
            _gather_start(ta_hbm, tb_hbm, ii_hbm, jj_hbm,
                          base + (2 * g + 2) * SCH, iv0, jv0,
                          bufa0, bufb0, sa0, sb0)

        _gather_finish(og_hbm, off_b, iv1, jv1, bufa1, bufb1, sa1, sb1)
        return carry

    lax.fori_loop(0, NHALF, outer, 0)


_sc_params = pltpu.CompilerParams(use_tc_tiling_on_sc=False)

_gather = pl.kernel(
    _gather_body,
    out_type=jax.ShapeDtypeStruct((E, TW), jnp.float32),
    mesh=_mesh,
    scratch_types=[
        pltpu.VMEM((SCH,), jnp.int32),
        pltpu.VMEM((SCH,), jnp.int32),
        pltpu.VMEM((SCH,), jnp.int32),
        pltpu.VMEM((SCH,), jnp.int32),
        pltpu.VMEM((SCH, TW), jnp.float32),
        pltpu.VMEM((SCH, TW), jnp.float32),
        pltpu.VMEM((SCH, TW), jnp.float32),
        pltpu.VMEM((SCH, TW), jnp.float32),
        pltpu.SemaphoreType.DMA,
        pltpu.SemaphoreType.DMA,
        pltpu.SemaphoreType.DMA,
        pltpu.SemaphoreType.DMA,
    ],
    compiler_params=_sc_params,
)


SD = 5
SGRP = NCHUNK // SD


def _scat_start(m_hbm, ii_hbm, off, iv, buf, sem):
    pltpu.async_copy(ii_hbm.at[pl.ds(off, CH)], iv, sem)
    pltpu.async_copy(m_hbm.at[pl.ds(off, CH)], buf, sem)


def _scat_flush(m_hbm, ii_hbm, acc, iv, buf, sem):
    pltpu.make_async_copy(ii_hbm.at[pl.ds(0, CH)], iv, sem).wait()
    pltpu.make_async_copy(m_hbm.at[pl.ds(0, CH)], buf, sem).wait()
    pltpu.sync_copy(buf, acc.at[iv], add=True)


def _scatter_body(m_hbm, ii_hbm, z_hbm, p_hbm,
                  iv0, iv1, iv2, iv3, iv4, b0, b1, b2, b3, b4, acc,
                  s0, s1, s2, s3, s4):
    c = lax.axis_index("c")
    s = lax.axis_index("s")
    wid = s * NC + c
    base = wid * EPW
    ivs = (iv0, iv1, iv2, iv3, iv4)
    bufs = (b0, b1, b2, b3, b4)
    sems = (s0, s1, s2, s3, s4)
    pltpu.sync_copy(z_hbm.at[pl.ds(s * RPT, RPT)], acc.at[pl.ds(s * RPT, RPT)])
    plsc.subcore_barrier()
    _scat_start(m_hbm, ii_hbm, base, ivs[0], bufs[0], sems[0])

    def outer(g, carry):
        for q in range(SD):
            nxt = SD * g + q + 1
            qn = (q + 1) % SD

            @pl.when(nxt < NCHUNK)
            def _():
                _scat_start(m_hbm, ii_hbm, base + nxt * CH,
                            ivs[qn], bufs[qn], sems[qn])

            _scat_flush(m_hbm, ii_hbm, acc, ivs[q], bufs[q], sems[q])
        return carry

    lax.fori_loop(0, SGRP, outer, 0)
    plsc.subcore_barrier()
    pltpu.sync_copy(acc.at[pl.ds(s * RPT, RPT)], p_hbm.at[c, pl.ds(s * RPT, RPT)])


_scatter = pl.kernel(
    _scatter_body,
    out_type=jax.ShapeDtypeStruct((NC, N, TW), jnp.float32),
    mesh=_mesh,
    scratch_types=(
        [pltpu.VMEM((CH,), jnp.int32)] * SD
        + [pltpu.VMEM((CH, TW), jnp.float32)] * SD
        + [pltpu.VMEM_SHARED((N, TW), jnp.float32)]
        + [pltpu.SemaphoreType.DMA] * SD
    ),
    compiler_params=_sc_params,
)


BN = 2000


def _node_pre_body(h_ref, x_ref, wa_ref, wb_ref, be1_ref, ta_ref, tb_ref):
    h = h_ref[...]
    x = x_ref[...]
    a = jnp.dot(h, wa_ref[...], preferred_element_type=jnp.float32) + be1_ref[...]
    b = jnp.dot(h, wb_ref[...], preferred_element_type=jnp.float32)
    ta_ref[...] = jnp.concatenate([a, x], axis=1)
    tb_ref[...] = jnp.concatenate([b, -x], axis=1)


_node_pre = pl.pallas_call(
    _node_pre_body,
    grid=(N // BN,),
    in_specs=[
        pl.BlockSpec((BN, ND), lambda i: (i, 0)),
        pl.BlockSpec((BN, XW), lambda i: (i, 0)),
        pl.BlockSpec((ND, HID), lambda i: (0, 0)),
        pl.BlockSpec((ND, HID), lambda i: (0, 0)),
        pl.BlockSpec((1, HID), lambda i: (0, 0)),
    ],
    out_specs=(pl.BlockSpec((BN, TW), lambda i: (i, 0)),
               pl.BlockSpec((BN, TW), lambda i: (i, 0))),
    out_shape=(jax.ShapeDtypeStruct((N, TW), jnp.float32),
               jax.ShapeDtypeStruct((N, TW), jnp.float32)),
)


BE = 2560


def _edge_mlp_body(g_ref, ea_ref, wea_ref, w256_ref, we2_ref, be2_ref,
                   wx1_ref, bx1_ref, wx2_ref, m_ref):
    g = g_ref[...]
    s = g[:, :HID]
    d = g[:, HID:]
    dsq = jnp.sum(d * d, axis=1, keepdims=True)
    dsqb = dsq.astype(jnp.bfloat16).astype(jnp.float32)
    w256b = w256_ref[...].astype(jnp.bfloat16).astype(jnp.float32)
    pre = (s + dsqb * w256b
           + jnp.dot(ea_ref[...], wea_ref[...], preferred_element_type=jnp.float32))
    m1 = _silu(pre)
    m2 = _silu(jnp.dot(m1, we2_ref[...], preferred_element_type=jnp.float32)
               + be2_ref[...])
    t = _silu(jnp.dot(m2, wx1_ref[...], preferred_element_type=jnp.float32)
              + bx1_ref[...])
    tb16 = t.astype(jnp.bfloat16).astype(jnp.float32)
    wxb = wx2_ref[...].astype(jnp.bfloat16).astype(jnp.float32)
    cw = jnp.sum(tb16 * wxb, axis=1, keepdims=True)
    m_ref[...] = jnp.concatenate([m2, d * cw], axis=1)


_edge_mlp = pl.pallas_call(
    _edge_mlp_body,
    grid=(E // BE,),
    in_specs=[
        pl.BlockSpec((BE, TW), lambda i: (i, 0)),
        pl.BlockSpec((BE, XW), lambda i: (i, 0)),
        pl.BlockSpec((XW, HID), lambda i: (0, 0)),
        pl.BlockSpec((1, HID), lambda i: (0, 0)),
        pl.BlockSpec((HID, HID), lambda i: (0, 0)),
        pl.BlockSpec((1, HID), lambda i: (0, 0)),
        pl.BlockSpec((HID, HID), lambda i: (0, 0)),
        pl.BlockSpec((1, HID), lambda i: (0, 0)),
        pl.BlockSpec((1, HID), lambda i: (0, 0)),
    ],
    out_specs=pl.BlockSpec((BE, TW), lambda i: (i, 0)),
    out_shape=jax.ShapeDtypeStruct((E, TW), jnp.float32),
)


def _node_fused_body(h_ref, x_ref, p0_ref, p1_ref, w1h_ref, w1m_ref, bh1_ref,
                     wh2_ref, bh2_ref, wa_ref, wb_ref, be1_ref,
                     ho_ref, xo_ref, ta_ref, tb_ref):
    h = h_ref[...]
    p0 = p0_ref[...]
    p1 = p1_ref[...]
    mi = p0[:, :HID] + p1[:, :HID]
    xu = p0[:, HID:] + p1[:, HID:]
    g = _silu(jnp.dot(h, w1h_ref[...], preferred_element_type=jnp.float32)
              + jnp.dot(mi, w1m_ref[...], preferred_element_type=jnp.float32)
              + bh1_ref[...])
    hn = h + jnp.dot(g, wh2_ref[...], preferred_element_type=jnp.float32) + bh2_ref[...]
    xn = x_ref[...] + xu
    ho_ref[...] = hn
    xo_ref[...] = xn
    a = jnp.dot(hn, wa_ref[...], preferred_element_type=jnp.float32) + be1_ref[...]
    b = jnp.dot(hn, wb_ref[...], preferred_element_type=jnp.float32)
    ta_ref[...] = jnp.concatenate([a, xn], axis=1)
    tb_ref[...] = jnp.concatenate([b, -xn], axis=1)


_node_fused = pl.pallas_call(
    _node_fused_body,
    grid=(N // BN,),
    in_specs=[
        pl.BlockSpec((BN, ND), lambda i: (i, 0)),
        pl.BlockSpec((BN, XW), lambda i: (i, 0)),
        pl.BlockSpec((BN, TW), lambda i: (i, 0)),
        pl.BlockSpec((BN, TW), lambda i: (i, 0)),
        pl.BlockSpec((ND, HID), lambda i: (0, 0)),
        pl.BlockSpec((HID, HID), lambda i: (0, 0)),
        pl.BlockSpec((1, HID), lambda i: (0, 0)),
        pl.BlockSpec((HID, ND), lambda i: (0, 0)),
        pl.BlockSpec((1, ND), lambda i: (0, 0)),
        pl.BlockSpec((ND, HID), lambda i: (0, 0)),
        pl.BlockSpec((ND, HID), lambda i: (0, 0)),
        pl.BlockSpec((1, HID), lambda i: (0, 0)),
    ],
    out_specs=(pl.BlockSpec((BN, ND), lambda i: (i, 0)),
               pl.BlockSpec((BN, XW), lambda i: (i, 0)),
               pl.BlockSpec((BN, TW), lambda i: (i, 0)),
               pl.BlockSpec((BN, TW), lambda i: (i, 0))),
    out_shape=(jax.ShapeDtypeStruct((N, ND), jnp.float32),
               jax.ShapeDtypeStruct((N, XW), jnp.float32),
               jax.ShapeDtypeStruct((N, TW), jnp.float32),
               jax.ShapeDtypeStruct((N, TW), jnp.float32)),
)


def _node_upd_body(h_ref, x_ref, p0_ref, p1_ref, w1h_ref, w1m_ref, bh1_ref,
                   wh2_ref, bh2_ref, ho_ref, xo_ref):
    h = h_ref[...]
    p0 = p0_ref[...]
    p1 = p1_ref[...]
    mi = p0[:, :HID] + p1[:, :HID]
    xu = p0[:, HID:] + p1[:, HID:]
    g = _silu(jnp.dot(h, w1h_ref[...], preferred_element_type=jnp.float32)
              + jnp.dot(mi, w1m_ref[...], preferred_element_type=jnp.float32)
              + bh1_ref[...])
    ho_ref[...] = h + jnp.dot(g, wh2_ref[...], preferred_element_type=jnp.float32) + bh2_ref[...]
    xo_ref[...] = x_ref[...] + xu


_node_upd = pl.pallas_call(
    _node_upd_body,
    grid=(N // BN,),
    in_specs=[
        pl.BlockSpec((BN, ND), lambda i: (i, 0)),
        pl.BlockSpec((BN, XW), lambda i: (i, 0)),
        pl.BlockSpec((BN, TW), lambda i: (i, 0)),
        pl.BlockSpec((BN, TW), lambda i: (i, 0)),
        pl.BlockSpec((ND, HID), lambda i: (0, 0)),
        pl.BlockSpec((HID, HID), lambda i: (0, 0)),
        pl.BlockSpec((1, HID), lambda i: (0, 0)),
        pl.BlockSpec((HID, ND), lambda i: (0, 0)),
        pl.BlockSpec((1, ND), lambda i: (0, 0)),
    ],
    out_specs=(pl.BlockSpec((BN, ND), lambda i: (i, 0)),
               pl.BlockSpec((BN, XW), lambda i: (i, 0))),
    out_shape=(jax.ShapeDtypeStruct((N, ND), jnp.float32),
               jax.ShapeDtypeStruct((N, XW), jnp.float32)),
)


def kernel(h, x, edge_index, edge_attr, We1, be1, We2, be2, Wx1, bx1, Wx2,
           Wh1, bh1, Wh2, bh2):
    ii = edge_index[1].astype(jnp.int32)
    jj = edge_index[0].astype(jnp.int32)
    xp = jnp.pad(x, ((0, 0), (0, XW - 3)))
    zacc = jnp.zeros((N, TW), jnp.float32)
    ta, tb = _node_pre(h, xp, We1[0, :ND], We1[0, ND:2 * ND], be1[0][None])
    for l in range(LAYERS):
        w256 = We1[l, 2 * ND:2 * ND + 1]
        wea = We1[l, 2 * ND + 1:]
        gg = _gather(ta, tb, ii, jj)
        m = _edge_mlp(gg, edge_attr, wea, w256, We2[l], be2[l][None],
                      Wx1[l], bx1[l][None], Wx2[l].T)
        p = _scatter(m, ii, zacc)
        if l < LAYERS - 1:
            h, xp, ta, tb = _node_fused(
                h, xp, p[0], p[1], Wh1[l, :ND], Wh1[l, ND:], bh1[l][None],
                Wh2[l], bh2[l][None], We1[l + 1, :ND], We1[l + 1, ND:2 * ND],
                be1[l + 1][None])
        else:
            h, xp = _node_upd(h, xp, p[0], p[1], Wh1[l, :ND], Wh1[l, ND:],
                              bh1[l][None], Wh2[l], bh2[l][None])
    return h, xp[:, :3]
